# Initial kernel scaffold; baseline (speedup 1.0000x reference)
import jax
import jax.numpy as jnp
from jax.experimental import pallas as pl


def kernel(x, edge_index, W_l1, b_l1, W_r1, W_l2, b_l2, W_r2):
    raise NotImplementedError("write your pallas kernel here")


# speedup vs baseline: 8.4533x; 8.4533x over previous
import functools

import jax
import jax.numpy as jnp
from jax import lax
from jax.experimental import pallas as pl
from jax.experimental.pallas import tpu as pltpu
from jax.experimental.pallas import tpu_sc as plsc

N = 10000
E = 320000
D = 128
N_PAD = 10240
SLICE = N_PAD // 16
CHUNK = 128
NCHUNK = E // CHUNK
NC, NS = 2, 16


def _sc_body(src_hbm, dst_hbm, x_hbm, zrows_hbm, zvec_hbm, ones_hbm,
             agg_out, g_out, inv_out,
             shared_agg, shared_cnt, shared_inv, shared_g,
             dstbuf, srcbuf, rowsbuf, onesbuf, wbuf, cntbuf, invbuf):
    tid = lax.axis_index("s")
    cid = lax.axis_index("c")
    wid = cid * NS + tid
    sl = pl.ds(tid * SLICE, SLICE)

    pltpu.sync_copy(ones_hbm, onesbuf)
    pltpu.sync_copy(zvec_hbm, shared_cnt.at[sl])
    pltpu.sync_copy(zvec_hbm, shared_g.at[sl])
    pltpu.sync_copy(zrows_hbm, shared_agg.at[sl])
    plsc.subcore_barrier()

    def p1(i, carry):
        c = tid + NS * i

        @pl.when(c < NCHUNK)
        def _():
            pltpu.sync_copy(dst_hbm.at[pl.ds(c * CHUNK, CHUNK)], dstbuf)
            pltpu.sync_copy(onesbuf, shared_cnt.at[dstbuf], add=True)
        return carry

    lax.fori_loop(0, (NCHUNK + NS - 1) // NS, p1, 0)
    plsc.subcore_barrier()

    pltpu.sync_copy(shared_cnt.at[sl], cntbuf)
    for j in range(SLICE // 16):
        v = cntbuf[pl.ds(j * 16, 16)]
        invbuf[pl.ds(j * 16, 16)] = 1.0 / jnp.maximum(v, 1.0)
    pltpu.sync_copy(invbuf, shared_inv.at[sl])
    pltpu.sync_copy(invbuf, inv_out.at[cid, sl])
    plsc.subcore_barrier()

    def p3(i, carry):
        c = wid + NC * NS * i

        @pl.when(c < NCHUNK)
        def _():
            off = c * CHUNK
            pltpu.sync_copy(dst_hbm.at[pl.ds(off, CHUNK)], dstbuf)
            pltpu.sync_copy(src_hbm.at[pl.ds(off, CHUNK)], srcbuf)
            pltpu.sync_copy(x_hbm.at[srcbuf], rowsbuf)
            pltpu.sync_copy(rowsbuf, shared_agg.at[dstbuf], add=True)
            pltpu.sync_copy(shared_inv.at[dstbuf], wbuf)
            pltpu.sync_copy(wbuf, shared_g.at[srcbuf], add=True)
        return carry

    lax.fori_loop(0, (NCHUNK + NC * NS - 1) // (NC * NS), p3, 0)
    plsc.subcore_barrier()

    pltpu.sync_copy(shared_agg.at[sl], agg_out.at[cid, sl])
    pltpu.sync_copy(shared_g.at[sl], g_out.at[cid, sl])


_sc_kernel = functools.partial(
    pl.kernel,
    out_type=(
        jax.ShapeDtypeStruct((NC, N_PAD, D), jnp.float32),
        jax.ShapeDtypeStruct((NC, N_PAD), jnp.float32),
        jax.ShapeDtypeStruct((NC, N_PAD), jnp.float32),
    ),
    mesh=plsc.VectorSubcoreMesh(
        core_axis_name="c", subcore_axis_name="s",
        num_cores=NC, num_subcores=NS),
    scratch_types=[
        pltpu.VMEM_SHARED((N_PAD, D), jnp.float32),
        pltpu.VMEM_SHARED((N_PAD,), jnp.float32),
        pltpu.VMEM_SHARED((N_PAD,), jnp.float32),
        pltpu.VMEM_SHARED((N_PAD,), jnp.float32),
        pltpu.VMEM((CHUNK,), jnp.int32),
        pltpu.VMEM((CHUNK,), jnp.int32),
        pltpu.VMEM((CHUNK, D), jnp.float32),
        pltpu.VMEM((CHUNK,), jnp.float32),
        pltpu.VMEM((CHUNK,), jnp.float32),
        pltpu.VMEM((SLICE,), jnp.float32),
        pltpu.VMEM((SLICE,), jnp.float32),
    ],
)(_sc_body)


def _tc_body(x_ref, aggp_ref, gp_ref, inv_ref,
             wl1t_ref, b1_ref, wr1t_ref, wl2t_ref, b2_ref, wr2t_ref,
             out_ref):
    agg = aggp_ref[0, :N, :] + aggp_ref[1, :N, :]
    inv = inv_ref[:N, :]
    mean1 = agg * inv
    x = x_ref[...]
    h = jnp.dot(mean1, wl1t_ref[...], preferred_element_type=jnp.float32)
    h += jnp.dot(x, wr1t_ref[...], preferred_element_type=jnp.float32)
    h += b1_ref[...]
    h = jnp.maximum(h, 0.0)
    g = gp_ref[0, :N, :] + gp_ref[1, :N, :]
    sh = jnp.sum(h, axis=0, keepdims=True)
    s2 = jnp.sum(h * g, axis=0, keepdims=True)
    pre = jnp.dot(s2, wl2t_ref[...], preferred_element_type=jnp.float32)
    pre += jnp.dot(sh, wr2t_ref[...], preferred_element_type=jnp.float32)
    pre = pre * (1.0 / N) + b2_ref[...]
    m = jnp.max(pre, axis=-1, keepdims=True)
    e = jnp.exp(pre - m)
    out_ref[...] = e / jnp.sum(e, axis=-1, keepdims=True)


_tc_kernel = pl.pallas_call(
    _tc_body,
    out_shape=jax.ShapeDtypeStruct((1, 16), jnp.float32),
)


def kernel(x, edge_index, W_l1, b_l1, W_r1, W_l2, b_l2, W_r2):
    src = edge_index[0]
    dst = edge_index[1]
    zrows = jnp.zeros((SLICE, D), jnp.float32)
    zvec = jnp.zeros((SLICE,), jnp.float32)
    ones = jnp.ones((CHUNK,), jnp.float32)
    agg_p, g_p, inv_p = _sc_kernel(src, dst, x, zrows, zvec, ones)
    return _tc_kernel(
        x, agg_p, g_p[:, :, None], inv_p[0][:, None],
        W_l1.T, b_l1[None, :], W_r1.T, W_l2.T, b_l2[None, :], W_r2.T)
